# Initial kernel scaffold; baseline (speedup 1.0000x reference)
import jax
import jax.numpy as jnp
from jax.experimental import pallas as pl


def kernel(x, W_router, W_up, W_down, b_up, b_down):
    raise NotImplementedError("write your pallas kernel here")


# speedup vs baseline: 5.7170x; 5.7170x over previous
import functools

import jax
import jax.numpy as jnp
from jax.experimental import pallas as pl
from jax.experimental.pallas import tpu as pltpu

B, S, D, U, E, K = 1, 256, 256, 512, 64, 2


def _moe_kernel(x_ref, wr_ref, wu_ref, wd_ref, bu_ref, bd_ref, out_ref,
                a1_ref, a2_ref, w1_ref, w2_ref):
    e = pl.program_id(0)

    @pl.when(e == 0)
    def _routing():
        x = x_ref[...]
        logits = jax.lax.dot_general(
            x, wr_ref[...], (((1,), (1,)), ((), ())),
            preferred_element_type=jnp.float32)
        ii = jax.lax.broadcasted_iota(jnp.int32, (S, E), 1)
        m1 = jnp.max(logits, axis=1, keepdims=True)
        a1 = jnp.min(jnp.where(logits == m1, ii, E), axis=1, keepdims=True)
        masked = jnp.where(ii == a1, -jnp.inf, logits)
        m2 = jnp.max(masked, axis=1, keepdims=True)
        a2 = jnp.min(jnp.where(masked == m2, ii, E), axis=1, keepdims=True)
        e2 = jnp.exp(m2 - m1)
        denom = 1.0 + e2
        a1_ref[...] = a1
        a2_ref[...] = a2
        w1_ref[...] = 1.0 / denom
        w2_ref[...] = e2 / denom
        out_ref[...] = jnp.zeros_like(out_ref)

    x = x_ref[...]
    wu = wu_ref[0]
    wd = wd_ref[0]
    h = jax.lax.dot_general(x, wu, (((1,), (1,)), ((), ())),
                            preferred_element_type=jnp.float32)
    h = h + bu_ref[0]
    h = 0.5 * h * (1.0 + jax.lax.erf(h * 0.7071067811865476))
    y = jax.lax.dot_general(h, wd, (((1,), (1,)), ((), ())),
                            preferred_element_type=jnp.float32)
    y = y + bd_ref[0]
    w = (jnp.where(a1_ref[...] == e, w1_ref[...], 0.0)
         + jnp.where(a2_ref[...] == e, w2_ref[...], 0.0))
    out_ref[...] += y * w


@jax.jit
def _moe(x2d, W_router, W_up, W_down, b_up, b_down):
    out = pl.pallas_call(
        _moe_kernel,
        grid=(E,),
        in_specs=[
            pl.BlockSpec((S, D), lambda e: (0, 0)),
            pl.BlockSpec((E, D), lambda e: (0, 0)),
            pl.BlockSpec((1, U, D), lambda e: (e, 0, 0)),
            pl.BlockSpec((1, D, U), lambda e: (e, 0, 0)),
            pl.BlockSpec((1, 1, U), lambda e: (e, 0, 0)),
            pl.BlockSpec((1, 1, D), lambda e: (e, 0, 0)),
        ],
        out_specs=pl.BlockSpec((S, D), lambda e: (0, 0)),
        out_shape=jax.ShapeDtypeStruct((S, D), jnp.float32),
        scratch_shapes=[
            pltpu.VMEM((S, 1), jnp.int32),
            pltpu.VMEM((S, 1), jnp.int32),
            pltpu.VMEM((S, 1), jnp.float32),
            pltpu.VMEM((S, 1), jnp.float32),
        ],
        compiler_params=pltpu.CompilerParams(
            dimension_semantics=("arbitrary",),
        ),
    )(x2d, W_router, W_up, W_down,
      b_up.reshape(E, 1, U), b_down.reshape(E, 1, D))
    return out


def kernel(x, W_router, W_up, W_down, b_up, b_down):
    out = _moe(x.reshape(S, D), W_router, W_up, W_down, b_up, b_down)
    return out.reshape(B, S, D)


# speedup vs baseline: 8.2697x; 1.4465x over previous
import functools

import jax
import jax.numpy as jnp
from jax.experimental import pallas as pl
from jax.experimental.pallas import tpu as pltpu

B, S, D, U, E, K = 1, 256, 256, 512, 64, 2
BE = 2


def _moe_kernel(x_ref, wr_ref, wu_ref, wd_ref, bu_ref, bd_ref, out_ref,
                a1_ref, a2_ref, w1_ref, w2_ref):
    e = pl.program_id(0)

    @pl.when(e == 0)
    def _routing():
        x = x_ref[...]
        logits = jax.lax.dot_general(
            x, wr_ref[...], (((1,), (1,)), ((), ())),
            preferred_element_type=jnp.float32)
        ii = jax.lax.broadcasted_iota(jnp.int32, (S, E), 1)
        m1 = jnp.max(logits, axis=1, keepdims=True)
        a1 = jnp.min(jnp.where(logits == m1, ii, E), axis=1, keepdims=True)
        masked = jnp.where(ii == a1, -jnp.inf, logits)
        m2 = jnp.max(masked, axis=1, keepdims=True)
        a2 = jnp.min(jnp.where(masked == m2, ii, E), axis=1, keepdims=True)
        e2 = jnp.exp(m2 - m1)
        denom = 1.0 + e2
        a1_ref[...] = a1
        a2_ref[...] = a2
        w1_ref[...] = 1.0 / denom
        w2_ref[...] = e2 / denom
        out_ref[...] = jnp.zeros_like(out_ref)

    x = x_ref[...]
    acc = jnp.zeros((S, D), jnp.float32)
    for be in range(BE):
        eid = e * BE + be
        wu = wu_ref[be]
        wd = wd_ref[be]
        h = jax.lax.dot_general(x, wu, (((1,), (1,)), ((), ())),
                                preferred_element_type=jnp.float32)
        h = h + bu_ref[be]
        h = 0.5 * h * (1.0 + jax.lax.erf(h * 0.7071067811865476))
        w = (jnp.where(a1_ref[...] == eid, w1_ref[...], 0.0)
             + jnp.where(a2_ref[...] == eid, w2_ref[...], 0.0))
        h = h * w
        y = jax.lax.dot_general(h, wd, (((1,), (1,)), ((), ())),
                                preferred_element_type=jnp.float32)
        acc += y + w * bd_ref[be]
    out_ref[...] += acc


@jax.jit
def _moe(x2d, W_router, W_up, W_down, b_up, b_down):
    out = pl.pallas_call(
        _moe_kernel,
        grid=(E // BE,),
        in_specs=[
            pl.BlockSpec((S, D), lambda e: (0, 0)),
            pl.BlockSpec((E, D), lambda e: (0, 0)),
            pl.BlockSpec((BE, U, D), lambda e: (e, 0, 0)),
            pl.BlockSpec((BE, D, U), lambda e: (e, 0, 0)),
            pl.BlockSpec((BE, 1, U), lambda e: (e, 0, 0)),
            pl.BlockSpec((BE, 1, D), lambda e: (e, 0, 0)),
        ],
        out_specs=pl.BlockSpec((S, D), lambda e: (0, 0)),
        out_shape=jax.ShapeDtypeStruct((S, D), jnp.float32),
        scratch_shapes=[
            pltpu.VMEM((S, 1), jnp.int32),
            pltpu.VMEM((S, 1), jnp.int32),
            pltpu.VMEM((S, 1), jnp.float32),
            pltpu.VMEM((S, 1), jnp.float32),
        ],
        compiler_params=pltpu.CompilerParams(
            dimension_semantics=("arbitrary",),
        ),
    )(x2d, W_router, W_up, W_down,
      b_up.reshape(E, 1, U), b_down.reshape(E, 1, D))
    return out


def kernel(x, W_router, W_up, W_down, b_up, b_down):
    out = _moe(x.reshape(S, D), W_router, W_up, W_down, b_up, b_down)
    return out.reshape(B, S, D)


# speedup vs baseline: 11.6424x; 1.4078x over previous
import functools

import jax
import jax.numpy as jnp
from jax.experimental import pallas as pl
from jax.experimental.pallas import tpu as pltpu

B, S, D, U, E, K = 1, 256, 256, 512, 64, 2
BE = 8


def _moe_kernel(x_ref, wr_ref, wu_ref, wd_ref, bu_ref, bd_ref, out_ref,
                a1_ref, a2_ref, w1_ref, w2_ref):
    e = pl.program_id(0)

    @pl.when(e == 0)
    def _routing():
        x = x_ref[...]
        logits = jax.lax.dot_general(
            x, wr_ref[...], (((1,), (1,)), ((), ())),
            preferred_element_type=jnp.float32)
        ii = jax.lax.broadcasted_iota(jnp.int32, (S, E), 1)
        m1 = jnp.max(logits, axis=1, keepdims=True)
        a1 = jnp.min(jnp.where(logits == m1, ii, E), axis=1, keepdims=True)
        masked = jnp.where(ii == a1, -jnp.inf, logits)
        m2 = jnp.max(masked, axis=1, keepdims=True)
        a2 = jnp.min(jnp.where(masked == m2, ii, E), axis=1, keepdims=True)
        e2 = jnp.exp(m2 - m1)
        denom = 1.0 + e2
        a1_ref[...] = a1
        a2_ref[...] = a2
        w1_ref[...] = 1.0 / denom
        w2_ref[...] = e2 / denom
        out_ref[...] = jnp.zeros_like(out_ref)

    x = x_ref[...]
    acc = jnp.zeros((S, D), jnp.float32)
    for be in range(BE):
        eid = e * BE + be
        wu = wu_ref[be]
        wd = wd_ref[be]
        h = jax.lax.dot_general(x, wu, (((1,), (1,)), ((), ())),
                                preferred_element_type=jnp.float32)
        h = h + bu_ref[be]
        h = 0.5 * h * (1.0 + jax.lax.erf(h * 0.7071067811865476))
        w = (jnp.where(a1_ref[...] == eid, w1_ref[...], 0.0)
             + jnp.where(a2_ref[...] == eid, w2_ref[...], 0.0))
        h = h * w
        y = jax.lax.dot_general(h, wd, (((1,), (1,)), ((), ())),
                                preferred_element_type=jnp.float32)
        acc += y + w * bd_ref[be]
    out_ref[...] += acc


@jax.jit
def _moe(x2d, W_router, W_up, W_down, b_up, b_down):
    out = pl.pallas_call(
        _moe_kernel,
        grid=(E // BE,),
        in_specs=[
            pl.BlockSpec((S, D), lambda e: (0, 0)),
            pl.BlockSpec((E, D), lambda e: (0, 0)),
            pl.BlockSpec((BE, U, D), lambda e: (e, 0, 0)),
            pl.BlockSpec((BE, D, U), lambda e: (e, 0, 0)),
            pl.BlockSpec((BE, 1, U), lambda e: (e, 0, 0)),
            pl.BlockSpec((BE, 1, D), lambda e: (e, 0, 0)),
        ],
        out_specs=pl.BlockSpec((S, D), lambda e: (0, 0)),
        out_shape=jax.ShapeDtypeStruct((S, D), jnp.float32),
        scratch_shapes=[
            pltpu.VMEM((S, 1), jnp.int32),
            pltpu.VMEM((S, 1), jnp.int32),
            pltpu.VMEM((S, 1), jnp.float32),
            pltpu.VMEM((S, 1), jnp.float32),
        ],
        compiler_params=pltpu.CompilerParams(
            dimension_semantics=("arbitrary",),
        ),
    )(x2d, W_router, W_up, W_down,
      b_up.reshape(E, 1, U), b_down.reshape(E, 1, D))
    return out


def kernel(x, W_router, W_up, W_down, b_up, b_down):
    out = _moe(x.reshape(S, D), W_router, W_up, W_down, b_up, b_down)
    return out.reshape(B, S, D)
